# baseline (speedup 1.0000x reference)
import functools

import jax
import jax.numpy as jnp
from jax import lax
from jax.experimental import pallas as pl
from jax.experimental.pallas import tpu as pltpu

N = 10000
E = 320000
DT = 0.01
RB = 2000
EB = 8000
EPS = 1e-5


def _ln(y, g, b):
    m = jnp.mean(y, axis=-1, keepdims=True)
    v = jnp.mean((y - m) * (y - m), axis=-1, keepdims=True)
    return (y - m) * lax.rsqrt(v + EPS) * g + b


def _dot(a, w):
    return jnp.dot(a, w, preferred_element_type=jnp.float32)


def _row_spec(cols):
    return pl.BlockSpec((RB, cols), lambda i: (i, 0))


def _erow_spec(cols):
    return pl.BlockSpec((EB, cols), lambda i: (i, 0))


def _full_spec(shape):
    nd = len(shape)
    return pl.BlockSpec(shape, lambda i: (0,) * nd)


def _tc_call(body, grid, in_arrays, in_specs, out_shapes, out_specs):
    return pl.pallas_call(
        body,
        grid=(grid,),
        in_specs=in_specs,
        out_specs=out_specs,
        out_shape=out_shapes,
    )(*in_arrays)


def _encode_body(x_ref, w1, b1, w2, b2, g, b, w1s, w1d, eb1,
                 h_ref, a_ref, bb_ref):
    x = x_ref[...]
    h = _ln(_dot(jax.nn.relu(_dot(x, w1[...]) + b1[...]), w2[...]) + b2[...],
            g[...], b[...])
    h_ref[...] = h
    a_ref[...] = _dot(h, w1s[...])
    bb_ref[...] = _dot(h, w1d[...]) + eb1[...]


def _edge0_body(ea_ref, gs_ref, gd_ref,
                ew1, eb1, ew2, eb2, eg, ebb,
                w1e, w2, b2, g, b,
                e_out):
    ea = ea_ref[...]
    e0 = _ln(_dot(jax.nn.relu(_dot(ea, ew1[...]) + eb1[...]), ew2[...]) + eb2[...],
             eg[...], ebb[...])
    u = jax.nn.relu(gs_ref[...] + gd_ref[...] + _dot(e0, w1e[...]))
    v = _ln(_dot(u, w2[...]) + b2[...], g[...], b[...])
    e_out[...] = e0 + v


def _edge1_body(e_ref, gs_ref, gd_ref,
                w1e, w2, b2, g, b,
                e_out):
    e0 = e_ref[...]
    u = jax.nn.relu(gs_ref[...] + gd_ref[...] + _dot(e0, w1e[...]))
    v = _ln(_dot(u, w2[...]) + b2[...], g[...], b[...])
    e_out[...] = e0 + v


def _node0_body(h_ref, p0_ref, p1_ref,
                w1a, w1b, b1, w2, b2, g, b,
                w1s, w1d, eb1,
                h_out, a_out, b_out):
    h = h_ref[...]
    agg = p0_ref[...] + p1_ref[...]
    u = jax.nn.relu(_dot(h, w1a[...]) + _dot(agg, w1b[...]) + b1[...])
    hn = h + _ln(_dot(u, w2[...]) + b2[...], g[...], b[...])
    h_out[...] = hn
    a_out[...] = _dot(hn, w1s[...])
    b_out[...] = _dot(hn, w1d[...]) + eb1[...]


def _final_body(h_ref, p0_ref, p1_ref, x_ref,
                w1a, w1b, b1, w2, b2, g, b,
                d1, db1, d2, db2,
                out_ref):
    h = h_ref[...]
    agg = p0_ref[...] + p1_ref[...]
    u = jax.nn.relu(_dot(h, w1a[...]) + _dot(agg, w1b[...]) + b1[...])
    h2 = h + _ln(_dot(u, w2[...]) + b2[...], g[...], b[...])
    o = _dot(jax.nn.relu(_dot(h2, d1[...]) + db1[...]), d2[...]) + db2[...]
    x = x_ref[...]
    mask = (x[:, 1:2] <= x[:, 2:3] + DT).astype(jnp.float32)
    out_ref[...] = o * mask


def _gather_ab(a_tab, b_tab, src, dst):
    return jnp.take(a_tab, src, axis=0), jnp.take(b_tab, dst, axis=0)


def _segment_partials(e, dst):
    agg = jax.ops.segment_sum(e, dst, num_segments=N)
    return agg, jnp.zeros_like(agg)


def kernel(x, edge_index, edge_attr, params):
    src = edge_index[0]
    dst = edge_index[1]
    enc_n, enc_e, dec = params["enc_n"], params["enc_e"], params["dec"]
    l0, l1 = params["layers"][0], params["layers"][1]

    def r2(a):
        return a.reshape(1, -1)

    f32 = jnp.float32
    sd64 = jax.ShapeDtypeStruct((N, 64), f32)
    se64 = jax.ShapeDtypeStruct((E, 64), f32)

    ew1s = {}
    for i, lp in enumerate((l0, l1)):
        w = lp["edge"]["W1"]
        ew1s[i] = (w[:64], w[64:128], w[128:])
    nw1s = {i: (lp["node"]["W1"][:64], lp["node"]["W1"][64:])
            for i, lp in enumerate((l0, l1))}

    h0, a0, b0 = _tc_call(
        _encode_body, N // RB,
        [x, enc_n["W1"], r2(enc_n["b1"]), enc_n["W2"], r2(enc_n["b2"]),
         r2(enc_n["g"]), r2(enc_n["b"]),
         ew1s[0][0], ew1s[0][1], r2(l0["edge"]["b1"])],
        [_row_spec(128)] + [_full_spec(s.shape) for s in
                            (enc_n["W1"], r2(enc_n["b1"]), enc_n["W2"],
                             r2(enc_n["b2"]), r2(enc_n["g"]), r2(enc_n["b"]),
                             ew1s[0][0], ew1s[0][1], r2(l0["edge"]["b1"]))],
        (sd64, sd64, sd64),
        (_row_spec(64), _row_spec(64), _row_spec(64)),
    )

    gs0, gd0 = _gather_ab(a0, b0, src, dst)

    e1 = _tc_call(
        _edge0_body, E // EB,
        [edge_attr, gs0, gd0,
         enc_e["W1"], r2(enc_e["b1"]), enc_e["W2"], r2(enc_e["b2"]),
         r2(enc_e["g"]), r2(enc_e["b"]),
         ew1s[0][2], l0["edge"]["W2"], r2(l0["edge"]["b2"]),
         r2(l0["edge"]["g"]), r2(l0["edge"]["b"])],
        [_erow_spec(2), _erow_spec(64), _erow_spec(64)] +
        [_full_spec(s.shape) for s in
         (enc_e["W1"], r2(enc_e["b1"]), enc_e["W2"], r2(enc_e["b2"]),
          r2(enc_e["g"]), r2(enc_e["b"]),
          ew1s[0][2], l0["edge"]["W2"], r2(l0["edge"]["b2"]),
          r2(l0["edge"]["g"]), r2(l0["edge"]["b"]))],
        se64, _erow_spec(64),
    )

    p0, p1 = _segment_partials(e1, dst)

    h1, a1, b1t = _tc_call(
        _node0_body, N // RB,
        [h0, p0, p1,
         nw1s[0][0], nw1s[0][1], r2(l0["node"]["b1"]),
         l0["node"]["W2"], r2(l0["node"]["b2"]),
         r2(l0["node"]["g"]), r2(l0["node"]["b"]),
         ew1s[1][0], ew1s[1][1], r2(l1["edge"]["b1"])],
        [_row_spec(64)] * 3 +
        [_full_spec(s.shape) for s in
         (nw1s[0][0], nw1s[0][1], r2(l0["node"]["b1"]),
          l0["node"]["W2"], r2(l0["node"]["b2"]),
          r2(l0["node"]["g"]), r2(l0["node"]["b"]),
          ew1s[1][0], ew1s[1][1], r2(l1["edge"]["b1"]))],
        (sd64, sd64, sd64),
        (_row_spec(64), _row_spec(64), _row_spec(64)),
    )

    gs1, gd1 = _gather_ab(a1, b1t, src, dst)

    e2 = _tc_call(
        _edge1_body, E // EB,
        [e1, gs1, gd1,
         ew1s[1][2], l1["edge"]["W2"], r2(l1["edge"]["b2"]),
         r2(l1["edge"]["g"]), r2(l1["edge"]["b"])],
        [_erow_spec(64)] * 3 +
        [_full_spec(s.shape) for s in
         (ew1s[1][2], l1["edge"]["W2"], r2(l1["edge"]["b2"]),
          r2(l1["edge"]["g"]), r2(l1["edge"]["b"]))],
        se64, _erow_spec(64),
    )

    q0, q1 = _segment_partials(e2, dst)

    out = _tc_call(
        _final_body, N // RB,
        [h1, q0, q1, x,
         nw1s[1][0], nw1s[1][1], r2(l1["node"]["b1"]),
         l1["node"]["W2"], r2(l1["node"]["b2"]),
         r2(l1["node"]["g"]), r2(l1["node"]["b"]),
         dec["W1"], r2(dec["b1"]), dec["W2"], r2(dec["b2"])],
        [_row_spec(64)] * 3 + [_row_spec(128)] +
        [_full_spec(s.shape) for s in
         (nw1s[1][0], nw1s[1][1], r2(l1["node"]["b1"]),
          l1["node"]["W2"], r2(l1["node"]["b2"]),
          r2(l1["node"]["g"]), r2(l1["node"]["b"]),
          dec["W1"], r2(dec["b1"]), dec["W2"], r2(dec["b2"]))],
        jax.ShapeDtypeStruct((N, 128), f32),
        _row_spec(128),
    )
    return out


# speedup vs baseline: 3.1427x; 3.1427x over previous
import functools

import jax
import jax.numpy as jnp
from jax import lax
from jax.experimental import pallas as pl
from jax.experimental.pallas import tpu as pltpu
from jax.experimental.pallas import tpu_sc as plsc

N = 10000
E = 320000
DT = 0.01
RB = 2000
EB = 8000
EPS = 1e-5

NC = 2
NS = 16
NW = NC * NS
EPW = E // NW
GC = 128
NFULL = EPW // GC
TAIL = EPW - NFULL * GC
ZR = N // NS


def _ln(y, g, b):
    m = jnp.mean(y, axis=-1, keepdims=True)
    v = jnp.mean((y - m) * (y - m), axis=-1, keepdims=True)
    return (y - m) * lax.rsqrt(v + EPS) * g + b


def _dot(a, w):
    return jnp.dot(a, w, preferred_element_type=jnp.float32)


def _row_spec(cols):
    return pl.BlockSpec((RB, cols), lambda i: (i, 0))


def _erow_spec(cols):
    return pl.BlockSpec((EB, cols), lambda i: (i, 0))


def _full_spec(shape):
    nd = len(shape)
    return pl.BlockSpec(shape, lambda i: (0,) * nd)


def _tc_call(body, grid, in_arrays, in_specs, out_shapes, out_specs):
    return pl.pallas_call(
        body,
        grid=(grid,),
        in_specs=in_specs,
        out_specs=out_specs,
        out_shape=out_shapes,
    )(*in_arrays)


def _encode_body(x_ref, w1, b1, w2, b2, g, b, w1s, w1d, eb1,
                 h_ref, a_ref, bb_ref):
    x = x_ref[...]
    h = _ln(_dot(jax.nn.relu(_dot(x, w1[...]) + b1[...]), w2[...]) + b2[...],
            g[...], b[...])
    h_ref[...] = h
    a_ref[...] = _dot(h, w1s[...])
    bb_ref[...] = _dot(h, w1d[...]) + eb1[...]


def _edge0_body(ea_ref, g_ref,
                ew1, eb1, ew2, eb2, eg, ebb,
                w1e, w2, b2, g, b,
                e_out):
    ea = ea_ref[...]
    e0 = _ln(_dot(jax.nn.relu(_dot(ea, ew1[...]) + eb1[...]), ew2[...]) + eb2[...],
             eg[...], ebb[...])
    u = jax.nn.relu(g_ref[...] + _dot(e0, w1e[...]))
    v = _ln(_dot(u, w2[...]) + b2[...], g[...], b[...])
    e_out[...] = e0 + v


def _edge1_body(e_ref, g_ref,
                w1e, w2, b2, g, b,
                e_out):
    e0 = e_ref[...]
    u = jax.nn.relu(g_ref[...] + _dot(e0, w1e[...]))
    v = _ln(_dot(u, w2[...]) + b2[...], g[...], b[...])
    e_out[...] = e0 + v


def _node0_body(h_ref, p0_ref, p1_ref,
                w1a, w1b, b1, w2, b2, g, b,
                w1s, w1d, eb1,
                h_out, a_out, b_out):
    h = h_ref[...]
    agg = p0_ref[...] + p1_ref[...]
    u = jax.nn.relu(_dot(h, w1a[...]) + _dot(agg, w1b[...]) + b1[...])
    hn = h + _ln(_dot(u, w2[...]) + b2[...], g[...], b[...])
    h_out[...] = hn
    a_out[...] = _dot(hn, w1s[...])
    b_out[...] = _dot(hn, w1d[...]) + eb1[...]


def _final_body(h_ref, p0_ref, p1_ref, x_ref,
                w1a, w1b, b1, w2, b2, g, b,
                d1, db1, d2, db2,
                out_ref):
    h = h_ref[...]
    agg = p0_ref[...] + p1_ref[...]
    u = jax.nn.relu(_dot(h, w1a[...]) + _dot(agg, w1b[...]) + b1[...])
    h2 = h + _ln(_dot(u, w2[...]) + b2[...], g[...], b[...])
    o = _dot(jax.nn.relu(_dot(h2, d1[...]) + db1[...]), d2[...]) + db2[...]
    x = x_ref[...]
    mask = (x[:, 1:2] <= x[:, 2:3] + DT).astype(jnp.float32)
    out_ref[...] = o * mask


def _sc_mesh():
    return plsc.VectorSubcoreMesh(core_axis_name="c", subcore_axis_name="s")


_SC_PARAMS = pltpu.CompilerParams(use_tc_tiling_on_sc=False)


def _gather_ab(a_tab, b_tab, src, dst):
    @functools.partial(
        pl.kernel,
        out_type=jax.ShapeDtypeStruct((E, 64), jnp.float32),
        mesh=_sc_mesh(),
        compiler_params=_SC_PARAMS,
        scratch_types=[
            pltpu.VMEM((EPW,), jnp.int32),
            pltpu.VMEM((EPW,), jnp.int32),
            pltpu.VMEM((GC, 64), jnp.float32),
            pltpu.VMEM((GC, 64), jnp.float32),
            pltpu.SemaphoreType.DMA,
            pltpu.SemaphoreType.DMA,
        ],
    )
    def gk(a_hbm, b_hbm, src_hbm, dst_hbm, g_hbm, sidx, didx, ra, rb, s1, s2):
        wid = lax.axis_index("c") * NS + lax.axis_index("s")
        base = wid * EPW
        pltpu.sync_copy(src_hbm.at[pl.ds(base, EPW)], sidx)
        pltpu.sync_copy(dst_hbm.at[pl.ds(base, EPW)], didx)

        def chunk(off, n):
            ca = pltpu.async_copy(a_hbm.at[sidx.at[pl.ds(off, n)]],
                                  ra.at[pl.ds(0, n)], s1)
            cb = pltpu.async_copy(b_hbm.at[didx.at[pl.ds(off, n)]],
                                  rb.at[pl.ds(0, n)], s2)
            ca.wait()
            cb.wait()

            def row(r, carry):
                for cc in range(4):
                    sl = pl.ds(cc * 16, 16)
                    ra[r, sl] = ra[r, sl] + rb[r, sl]
                return carry

            lax.fori_loop(0, n, row, 0)
            pltpu.sync_copy(ra.at[pl.ds(0, n)], g_hbm.at[pl.ds(base + off, n)])

        def body(j, carry):
            chunk(j * GC, GC)
            return carry

        lax.fori_loop(0, NFULL, body, 0)
        if TAIL:
            chunk(NFULL * GC, TAIL)

    return gk(a_tab, b_tab, src, dst)


def _segment_partials(e, dst):
    @functools.partial(
        pl.kernel,
        out_type=jax.ShapeDtypeStruct((NC, N, 64), jnp.float32),
        mesh=_sc_mesh(),
        compiler_params=_SC_PARAMS,
        scratch_types=[
            pltpu.VMEM((GC,), jnp.int32),
            pltpu.VMEM((TAIL,), jnp.int32) if TAIL else None,
            pltpu.VMEM((GC, 64), jnp.float32),
            pltpu.VMEM((TAIL, 64), jnp.float32) if TAIL else None,
            pltpu.VMEM((ZR, 64), jnp.float32),
            pltpu.VMEM_SHARED((N, 64), jnp.float32),
        ],
    )
    def sk(e_hbm, dst_hbm, out_hbm, idxc, idxt, ebuf, ebuft, zbuf, agg):
        cid = lax.axis_index("c")
        sid = lax.axis_index("s")
        base = (cid * NS + sid) * EPW

        def zrow(r, carry):
            for cc in range(4):
                zbuf[r, pl.ds(cc * 16, 16)] = jnp.zeros((16,), jnp.float32)
            return carry

        lax.fori_loop(0, ZR, zrow, 0)
        pltpu.sync_copy(zbuf, agg.at[pl.ds(sid * ZR, ZR)])
        plsc.subcore_barrier()

        def body(j, carry):
            off = base + j * GC
            pltpu.sync_copy(dst_hbm.at[pl.ds(off, GC)], idxc)
            pltpu.sync_copy(e_hbm.at[pl.ds(off, GC)], ebuf)
            pltpu.sync_copy(ebuf, agg.at[idxc], add=True)
            return carry

        lax.fori_loop(0, NFULL, body, 0)
        if TAIL:
            off = base + NFULL * GC
            pltpu.sync_copy(dst_hbm.at[pl.ds(off, TAIL)], idxt)
            pltpu.sync_copy(e_hbm.at[pl.ds(off, TAIL)], ebuft)
            pltpu.sync_copy(ebuft, agg.at[idxt], add=True)

        plsc.subcore_barrier()

        @pl.when(sid == 0)
        def _():
            pltpu.sync_copy(agg, out_hbm.at[cid])

    parts = sk(e, dst)
    return parts[0], parts[1]


def kernel(x, edge_index, edge_attr, params):
    src = edge_index[0]
    dst = edge_index[1]
    enc_n, enc_e, dec = params["enc_n"], params["enc_e"], params["dec"]
    l0, l1 = params["layers"][0], params["layers"][1]

    def r2(a):
        return a.reshape(1, -1)

    f32 = jnp.float32
    sd64 = jax.ShapeDtypeStruct((N, 64), f32)
    se64 = jax.ShapeDtypeStruct((E, 64), f32)

    ew1s = {}
    for i, lp in enumerate((l0, l1)):
        w = lp["edge"]["W1"]
        ew1s[i] = (w[:64], w[64:128], w[128:])
    nw1s = {i: (lp["node"]["W1"][:64], lp["node"]["W1"][64:])
            for i, lp in enumerate((l0, l1))}

    h0, a0, b0 = _tc_call(
        _encode_body, N // RB,
        [x, enc_n["W1"], r2(enc_n["b1"]), enc_n["W2"], r2(enc_n["b2"]),
         r2(enc_n["g"]), r2(enc_n["b"]),
         ew1s[0][0], ew1s[0][1], r2(l0["edge"]["b1"])],
        [_row_spec(128)] + [_full_spec(s.shape) for s in
                            (enc_n["W1"], r2(enc_n["b1"]), enc_n["W2"],
                             r2(enc_n["b2"]), r2(enc_n["g"]), r2(enc_n["b"]),
                             ew1s[0][0], ew1s[0][1], r2(l0["edge"]["b1"]))],
        (sd64, sd64, sd64),
        (_row_spec(64), _row_spec(64), _row_spec(64)),
    )

    g0 = _gather_ab(a0, b0, src, dst)

    e1 = _tc_call(
        _edge0_body, E // EB,
        [edge_attr, g0,
         enc_e["W1"], r2(enc_e["b1"]), enc_e["W2"], r2(enc_e["b2"]),
         r2(enc_e["g"]), r2(enc_e["b"]),
         ew1s[0][2], l0["edge"]["W2"], r2(l0["edge"]["b2"]),
         r2(l0["edge"]["g"]), r2(l0["edge"]["b"])],
        [_erow_spec(2), _erow_spec(64)] +
        [_full_spec(s.shape) for s in
         (enc_e["W1"], r2(enc_e["b1"]), enc_e["W2"], r2(enc_e["b2"]),
          r2(enc_e["g"]), r2(enc_e["b"]),
          ew1s[0][2], l0["edge"]["W2"], r2(l0["edge"]["b2"]),
          r2(l0["edge"]["g"]), r2(l0["edge"]["b"]))],
        se64, _erow_spec(64),
    )

    p0, p1 = _segment_partials(e1, dst)

    h1, a1, b1t = _tc_call(
        _node0_body, N // RB,
        [h0, p0, p1,
         nw1s[0][0], nw1s[0][1], r2(l0["node"]["b1"]),
         l0["node"]["W2"], r2(l0["node"]["b2"]),
         r2(l0["node"]["g"]), r2(l0["node"]["b"]),
         ew1s[1][0], ew1s[1][1], r2(l1["edge"]["b1"])],
        [_row_spec(64)] * 3 +
        [_full_spec(s.shape) for s in
         (nw1s[0][0], nw1s[0][1], r2(l0["node"]["b1"]),
          l0["node"]["W2"], r2(l0["node"]["b2"]),
          r2(l0["node"]["g"]), r2(l0["node"]["b"]),
          ew1s[1][0], ew1s[1][1], r2(l1["edge"]["b1"]))],
        (sd64, sd64, sd64),
        (_row_spec(64), _row_spec(64), _row_spec(64)),
    )

    g1 = _gather_ab(a1, b1t, src, dst)

    e2 = _tc_call(
        _edge1_body, E // EB,
        [e1, g1,
         ew1s[1][2], l1["edge"]["W2"], r2(l1["edge"]["b2"]),
         r2(l1["edge"]["g"]), r2(l1["edge"]["b"])],
        [_erow_spec(64)] * 2 +
        [_full_spec(s.shape) for s in
         (ew1s[1][2], l1["edge"]["W2"], r2(l1["edge"]["b2"]),
          r2(l1["edge"]["g"]), r2(l1["edge"]["b"]))],
        se64, _erow_spec(64),
    )

    q0, q1 = _segment_partials(e2, dst)

    out = _tc_call(
        _final_body, N // RB,
        [h1, q0, q1, x,
         nw1s[1][0], nw1s[1][1], r2(l1["node"]["b1"]),
         l1["node"]["W2"], r2(l1["node"]["b2"]),
         r2(l1["node"]["g"]), r2(l1["node"]["b"]),
         dec["W1"], r2(dec["b1"]), dec["W2"], r2(dec["b2"])],
        [_row_spec(64)] * 3 + [_row_spec(128)] +
        [_full_spec(s.shape) for s in
         (nw1s[1][0], nw1s[1][1], r2(l1["node"]["b1"]),
          l1["node"]["W2"], r2(l1["node"]["b2"]),
          r2(l1["node"]["g"]), r2(l1["node"]["b"]),
          dec["W1"], r2(dec["b1"]), dec["W2"], r2(dec["b2"]))],
        jax.ShapeDtypeStruct((N, 128), f32),
        _row_spec(128),
    )
    return out


# speedup vs baseline: 4.5924x; 1.4613x over previous
import functools

import jax
import jax.numpy as jnp
from jax import lax
from jax.experimental import pallas as pl
from jax.experimental.pallas import tpu as pltpu
from jax.experimental.pallas import tpu_sc as plsc

N = 10000
E = 320000
DT = 0.01
RB = 2000
EBP = 4000
EPS = 1e-5

NC = 2
NS = 16
NW = NC * NS
EPW = E // NW
GC = 128
NFULL = EPW // GC
TAIL = EPW - NFULL * GC
ZR = N // NS


def _ln(y, g, b):
    m = jnp.mean(y, axis=-1, keepdims=True)
    v = jnp.mean((y - m) * (y - m), axis=-1, keepdims=True)
    return (y - m) * lax.rsqrt(v + EPS) * g + b


def _ln_p(y, g, b, gsum, gbc):
    m = _dot(_dot(y, gsum) * (1.0 / 64.0), gbc)
    d = y - m
    v = _dot(_dot(d * d, gsum) * (1.0 / 64.0), gbc)
    return d * lax.rsqrt(v + EPS) * g + b


def _dot(a, w):
    return jnp.dot(a, w, preferred_element_type=jnp.float32)


def _row_spec(cols):
    return pl.BlockSpec((RB, cols), lambda i: (i, 0))


def _erow_spec(cols):
    return pl.BlockSpec((EBP, cols), lambda i: (i, 0))


def _full_spec(shape):
    nd = len(shape)
    return pl.BlockSpec(shape, lambda i: (0,) * nd)


def _tc_call(body, grid, in_arrays, in_specs, out_shapes, out_specs):
    return pl.pallas_call(
        body,
        grid=(grid,),
        in_specs=in_specs,
        out_specs=out_specs,
        out_shape=out_shapes,
    )(*in_arrays)


def _encode_body(x_ref, w1, b1, w2, b2, g, b, w1s, w1d, eb1,
                 h_ref, a_ref, bb_ref):
    x = x_ref[...]
    h = _ln(_dot(jax.nn.relu(_dot(x, w1[...]) + b1[...]), w2[...]) + b2[...],
            g[...], b[...])
    h_ref[...] = h
    a_ref[...] = _dot(h, w1s[...])
    bb_ref[...] = _dot(h, w1d[...]) + eb1[...]


def _edge0_body(ea_ref, g_ref,
                ew1, eb1, ew2, eb2, eg, ebb,
                w1e, w2, b2, g, b, gsum, gbc,
                e_out):
    ea = ea_ref[...]
    e0 = _ln_p(_dot(jax.nn.relu(_dot(ea, ew1[...]) + eb1[...]), ew2[...]) + eb2[...],
               eg[...], ebb[...], gsum[...], gbc[...])
    u = jax.nn.relu(g_ref[...] + _dot(e0, w1e[...]))
    v = _ln_p(_dot(u, w2[...]) + b2[...], g[...], b[...], gsum[...], gbc[...])
    e_out[...] = e0 + v


def _edge1_body(e_ref, g_ref,
                w1e, w2, b2, g, b, gsum, gbc,
                e_out):
    e0 = e_ref[...]
    u = jax.nn.relu(g_ref[...] + _dot(e0, w1e[...]))
    v = _ln_p(_dot(u, w2[...]) + b2[...], g[...], b[...], gsum[...], gbc[...])
    e_out[...] = e0 + v


def _node0_body(h_ref, p0_ref, p1_ref,
                w1a, w1b, b1, w2, b2, g, b,
                w1s, w1d, eb1,
                h_out, a_out, b_out):
    h = h_ref[...]
    agg = p0_ref[...] + p1_ref[...]
    u = jax.nn.relu(_dot(h, w1a[...]) + _dot(agg, w1b[...]) + b1[...])
    hn = h + _ln(_dot(u, w2[...]) + b2[...], g[...], b[...])
    h_out[...] = hn
    a_out[...] = _dot(hn, w1s[...])
    b_out[...] = _dot(hn, w1d[...]) + eb1[...]


def _final_body(h_ref, p0_ref, p1_ref, x_ref,
                w1a, w1b, b1, w2, b2, g, b,
                d1, db1, d2, db2,
                out_ref):
    h = h_ref[...]
    agg = p0_ref[...] + p1_ref[...]
    u = jax.nn.relu(_dot(h, w1a[...]) + _dot(agg, w1b[...]) + b1[...])
    h2 = h + _ln(_dot(u, w2[...]) + b2[...], g[...], b[...])
    o = _dot(jax.nn.relu(_dot(h2, d1[...]) + db1[...]), d2[...]) + db2[...]
    x = x_ref[...]
    mask = (x[:, 1:2] <= x[:, 2:3] + DT).astype(jnp.float32)
    out_ref[...] = o * mask


def _sc_mesh():
    return plsc.VectorSubcoreMesh(core_axis_name="c", subcore_axis_name="s")


_SC_PARAMS = pltpu.CompilerParams(use_tc_tiling_on_sc=False)


def _gather_ab(a_tab, b_tab, src, dst):
    @functools.partial(
        pl.kernel,
        out_type=jax.ShapeDtypeStruct((E // 2, 128), jnp.float32),
        mesh=_sc_mesh(),
        compiler_params=_SC_PARAMS,
        scratch_types=[
            pltpu.VMEM((EPW,), jnp.int32),
            pltpu.VMEM((EPW,), jnp.int32),
            pltpu.VMEM((GC, 64), jnp.float32),
            pltpu.VMEM((GC, 64), jnp.float32),
            pltpu.VMEM((GC // 2, 128), jnp.float32),
            pltpu.SemaphoreType.DMA,
            pltpu.SemaphoreType.DMA,
        ],
    )
    def gk(a_hbm, b_hbm, src_hbm, dst_hbm, g_hbm, sidx, didx, ra, rb, rc, s1, s2):
        wid = lax.axis_index("c") * NS + lax.axis_index("s")
        base = wid * EPW
        pltpu.sync_copy(src_hbm.at[pl.ds(base, EPW)], sidx)
        pltpu.sync_copy(dst_hbm.at[pl.ds(base, EPW)], didx)

        def chunk(off, n):
            ca = pltpu.async_copy(a_hbm.at[sidx.at[pl.ds(off, n)]],
                                  ra.at[pl.ds(0, n)], s1)
            cb = pltpu.async_copy(b_hbm.at[didx.at[pl.ds(off, n)]],
                                  rb.at[pl.ds(0, n)], s2)
            ca.wait()
            cb.wait()

            def row(j, carry):
                for par in range(2):
                    for cc in range(4):
                        sl = pl.ds(cc * 16, 16)
                        rc[j, pl.ds(par * 64 + cc * 16, 16)] = (
                            ra[2 * j + par, sl] + rb[2 * j + par, sl])
                return carry

            lax.fori_loop(0, n // 2, row, 0)
            pltpu.sync_copy(rc.at[pl.ds(0, n // 2)],
                            g_hbm.at[pl.ds((base + off) // 2, n // 2)])

        def body(j, carry):
            chunk(j * GC, GC)
            return carry

        lax.fori_loop(0, NFULL, body, 0)
        if TAIL:
            chunk(NFULL * GC, TAIL)

    return gk(a_tab, b_tab, src, dst)


def _segment_partials(e, dst):
    @functools.partial(
        pl.kernel,
        out_type=jax.ShapeDtypeStruct((NC, N, 64), jnp.float32),
        mesh=_sc_mesh(),
        compiler_params=_SC_PARAMS,
        scratch_types=[
            pltpu.VMEM((GC,), jnp.int32),
            pltpu.VMEM((TAIL,), jnp.int32) if TAIL else None,
            pltpu.VMEM((GC, 64), jnp.float32),
            pltpu.VMEM((TAIL, 64), jnp.float32) if TAIL else None,
            pltpu.VMEM((GC // 2, 128), jnp.float32),
            pltpu.VMEM((ZR, 64), jnp.float32),
            pltpu.VMEM_SHARED((N, 64), jnp.float32),
        ],
    )
    def sk(e_hbm, dst_hbm, out_hbm, idxc, idxt, ebuf, ebuft, epk, zbuf, agg):
        cid = lax.axis_index("c")
        sid = lax.axis_index("s")
        base = (cid * NS + sid) * EPW

        def zrow(r, carry):
            for cc in range(4):
                zbuf[r, pl.ds(cc * 16, 16)] = jnp.zeros((16,), jnp.float32)
            return carry

        lax.fori_loop(0, ZR, zrow, 0)
        pltpu.sync_copy(zbuf, agg.at[pl.ds(sid * ZR, ZR)])
        plsc.subcore_barrier()

        def unpack(nrows):
            def row(j, carry):
                for par in range(2):
                    for cc in range(4):
                        sl = pl.ds(cc * 16, 16)
                        ebuf[2 * j + par, sl] = epk[j, pl.ds(par * 64 + cc * 16, 16)]
                return carry
            lax.fori_loop(0, nrows, row, 0)

        def body(j, carry):
            off = base + j * GC
            pltpu.sync_copy(dst_hbm.at[pl.ds(off, GC)], idxc)
            pltpu.sync_copy(e_hbm.at[pl.ds(off // 2, GC // 2)], epk.at[pl.ds(0, GC // 2)])
            unpack(GC // 2)
            pltpu.sync_copy(ebuf, agg.at[idxc], add=True)
            return carry

        lax.fori_loop(0, NFULL, body, 0)
        if TAIL:
            off = base + NFULL * GC
            pltpu.sync_copy(dst_hbm.at[pl.ds(off, TAIL)], idxt)
            pltpu.sync_copy(e_hbm.at[pl.ds(off // 2, TAIL // 2)], epk.at[pl.ds(0, TAIL // 2)])

            def trow(j, carry):
                for par in range(2):
                    for cc in range(4):
                        sl = pl.ds(cc * 16, 16)
                        ebuft[2 * j + par, sl] = epk[j, pl.ds(par * 64 + cc * 16, 16)]
                return carry

            lax.fori_loop(0, TAIL // 2, trow, 0)
            pltpu.sync_copy(ebuft, agg.at[idxt], add=True)

        plsc.subcore_barrier()

        @pl.when(sid == 0)
        def _():
            pltpu.sync_copy(agg, out_hbm.at[cid])

    parts = sk(e, dst)
    return parts[0], parts[1]


def kernel(x, edge_index, edge_attr, params):
    src = edge_index[0]
    dst = edge_index[1]
    enc_n, enc_e, dec = params["enc_n"], params["enc_e"], params["dec"]
    l0, l1 = params["layers"][0], params["layers"][1]

    def r2(a):
        return a.reshape(1, -1)

    f32 = jnp.float32
    sd64 = jax.ShapeDtypeStruct((N, 64), f32)
    se64 = jax.ShapeDtypeStruct((E, 64), f32)

    ew1s = {}
    for i, lp in enumerate((l0, l1)):
        w = lp["edge"]["W1"]
        ew1s[i] = (w[:64], w[64:128], w[128:])
    nw1s = {i: (lp["node"]["W1"][:64], lp["node"]["W1"][64:])
            for i, lp in enumerate((l0, l1))}

    h0, a0, b0 = _tc_call(
        _encode_body, N // RB,
        [x, enc_n["W1"], r2(enc_n["b1"]), enc_n["W2"], r2(enc_n["b2"]),
         r2(enc_n["g"]), r2(enc_n["b"]),
         ew1s[0][0], ew1s[0][1], r2(l0["edge"]["b1"])],
        [_row_spec(128)] + [_full_spec(s.shape) for s in
                            (enc_n["W1"], r2(enc_n["b1"]), enc_n["W2"],
                             r2(enc_n["b2"]), r2(enc_n["g"]), r2(enc_n["b"]),
                             ew1s[0][0], ew1s[0][1], r2(l0["edge"]["b1"]))],
        (sd64, sd64, sd64),
        (_row_spec(64), _row_spec(64), _row_spec(64)),
    )

    def bd(w):
        z = jnp.zeros_like(w)
        return jnp.concatenate(
            [jnp.concatenate([w, z], axis=1), jnp.concatenate([z, w], axis=1)],
            axis=0)

    def p2(v):
        return jnp.concatenate([v, v]).reshape(1, 128)

    gsum = jnp.concatenate(
        [jnp.concatenate([jnp.ones((64, 1), f32), jnp.zeros((64, 1), f32)], axis=1),
         jnp.concatenate([jnp.zeros((64, 1), f32), jnp.ones((64, 1), f32)], axis=1)],
        axis=0)
    gbc = gsum.T
    eap = edge_attr.reshape(E // 2, 4)
    sep = jax.ShapeDtypeStruct((E // 2, 128), f32)

    g0 = _gather_ab(a0, b0, src, dst)

    ew0 = (bd(enc_e["W1"]), p2(enc_e["b1"]), bd(enc_e["W2"]), p2(enc_e["b2"]),
           p2(enc_e["g"]), p2(enc_e["b"]),
           bd(ew1s[0][2]), bd(l0["edge"]["W2"]), p2(l0["edge"]["b2"]),
           p2(l0["edge"]["g"]), p2(l0["edge"]["b"]), gsum, gbc)
    e1 = _tc_call(
        _edge0_body, E // 2 // EBP,
        [eap, g0] + list(ew0),
        [_erow_spec(4), _erow_spec(128)] +
        [_full_spec(s.shape) for s in ew0],
        sep, _erow_spec(128),
    )

    p0, p1 = _segment_partials(e1, dst)

    h1, a1, b1t = _tc_call(
        _node0_body, N // RB,
        [h0, p0, p1,
         nw1s[0][0], nw1s[0][1], r2(l0["node"]["b1"]),
         l0["node"]["W2"], r2(l0["node"]["b2"]),
         r2(l0["node"]["g"]), r2(l0["node"]["b"]),
         ew1s[1][0], ew1s[1][1], r2(l1["edge"]["b1"])],
        [_row_spec(64)] * 3 +
        [_full_spec(s.shape) for s in
         (nw1s[0][0], nw1s[0][1], r2(l0["node"]["b1"]),
          l0["node"]["W2"], r2(l0["node"]["b2"]),
          r2(l0["node"]["g"]), r2(l0["node"]["b"]),
          ew1s[1][0], ew1s[1][1], r2(l1["edge"]["b1"]))],
        (sd64, sd64, sd64),
        (_row_spec(64), _row_spec(64), _row_spec(64)),
    )

    g1 = _gather_ab(a1, b1t, src, dst)

    ew1 = (bd(ew1s[1][2]), bd(l1["edge"]["W2"]), p2(l1["edge"]["b2"]),
           p2(l1["edge"]["g"]), p2(l1["edge"]["b"]), gsum, gbc)
    e2 = _tc_call(
        _edge1_body, E // 2 // EBP,
        [e1, g1] + list(ew1),
        [_erow_spec(128)] * 2 +
        [_full_spec(s.shape) for s in ew1],
        sep, _erow_spec(128),
    )

    q0, q1 = _segment_partials(e2, dst)

    out = _tc_call(
        _final_body, N // RB,
        [h1, q0, q1, x,
         nw1s[1][0], nw1s[1][1], r2(l1["node"]["b1"]),
         l1["node"]["W2"], r2(l1["node"]["b2"]),
         r2(l1["node"]["g"]), r2(l1["node"]["b"]),
         dec["W1"], r2(dec["b1"]), dec["W2"], r2(dec["b2"])],
        [_row_spec(64)] * 3 + [_row_spec(128)] +
        [_full_spec(s.shape) for s in
         (nw1s[1][0], nw1s[1][1], r2(l1["node"]["b1"]),
          l1["node"]["W2"], r2(l1["node"]["b2"]),
          r2(l1["node"]["g"]), r2(l1["node"]["b"]),
          dec["W1"], r2(dec["b1"]), dec["W2"], r2(dec["b2"]))],
        jax.ShapeDtypeStruct((N, 128), f32),
        _row_spec(128),
    )
    return out
